# baseline (speedup 1.0000x reference)
import functools

import jax
import jax.numpy as jnp
from jax import lax
from jax.experimental import pallas as pl
from jax.experimental.pallas import tpu as pltpu
from jax.experimental.pallas import tpu_sc as plsc

_B, _S, _D = 4, 2048, 1024
_NC, _NS = 2, 16
_NW = _NC * _NS
_SPW = _S // _NW
_SUB = 16
_NSUB = _SPW // _SUB
_STEPS = _NSUB * _B
_NXB = 6
_AHEAD = 4
_NEB = 1
_VECS = _SUB * _D // 16

_mesh = plsc.VectorSubcoreMesh(core_axis_name="c", subcore_axis_name="s")


@functools.partial(
    pl.kernel,
    mesh=_mesh,
    out_type=jax.ShapeDtypeStruct((_B, _S, _D), jnp.float32),
    compiler_params=pltpu.CompilerParams(use_tc_tiling_on_sc=True),
    scratch_types=[
        pltpu.VMEM((_NXB, _SUB, _D), jnp.float32),
        pltpu.VMEM((_NEB, _SUB, _D), jnp.float32),
        pltpu.SemaphoreType.DMA((_NXB,)),
        pltpu.SemaphoreType.DMA((_NXB,)),
        pltpu.SemaphoreType.DMA((_NEB,)),
    ],
)
def _sc_add(x_hbm, emb_hbm, out_hbm, xbuf, ebuf, xsem, osem, esem):
    wid = lax.axis_index("s") * _NC + lax.axis_index("c")
    s0 = wid * _SPW

    def rows(j):
        return pl.ds(s0 + j * _SUB, _SUB)

    def fire_x(t):
        j, b = divmod(t, _B)
        k = t % _NXB
        return pltpu.async_copy(x_hbm.at[b, rows(j)], xbuf.at[k], xsem.at[k])

    def fire_e(j):
        je = j % _NEB
        return pltpu.async_copy(emb_hbm.at[rows(j)], ebuf.at[je], esem.at[je])

    x_copies = {t: fire_x(t) for t in range(_AHEAD)}
    e_copies = {j: fire_e(j) for j in range(_NEB)}
    out_copies = {}

    for t in range(_STEPS):
        j, b = divmod(t, _B)
        k = t % _NXB
        if b == 0:
            e_copies.pop(j).wait()
        x_copies.pop(t).wait()
        t3 = t + _AHEAD
        if t3 < _STEPS:
            if t3 - _NXB in out_copies:
                out_copies.pop(t3 - _NXB).wait()
            x_copies[t3] = fire_x(t3)

        je = j % _NEB

        @plsc.parallel_loop(0, _VECS, unroll=8)
        def _add(i):
            r = i >> 6
            c = (i & 63) * 16
            plsc.addupdate(xbuf.at[k, r, pl.ds(c, 16)], ebuf[je, r, pl.ds(c, 16)])

        out_copies[t] = pltpu.async_copy(
            xbuf.at[k], out_hbm.at[b, rows(j)], osem.at[k])
        if b == _B - 1 and j + _NEB < _NSUB:
            e_copies[j + _NEB] = fire_e(j + _NEB)

    for t in sorted(out_copies):
        out_copies.pop(t).wait()


def kernel(x, emb):
    return _sc_add(x, emb)


# speedup vs baseline: 1.0711x; 1.0711x over previous
import functools

import jax
import jax.numpy as jnp
from jax import lax
from jax.experimental import pallas as pl
from jax.experimental.pallas import tpu as pltpu
from jax.experimental.pallas import tpu_sc as plsc

_B, _S, _D = 4, 2048, 1024
_NC, _NS = 2, 16
_NW = _NC * _NS
_SPW = _S // _NW
_SUB = 16
_NSUB = _SPW // _SUB
_STEPS = _NSUB * _B
_NXB = 5
_AHEAD = 3
_NEB = 2
_VECS = _SUB * _D // 16

_mesh = plsc.VectorSubcoreMesh(core_axis_name="c", subcore_axis_name="s")


@functools.partial(
    pl.kernel,
    mesh=_mesh,
    out_type=jax.ShapeDtypeStruct((_B, _S, _D), jnp.float32),
    compiler_params=pltpu.CompilerParams(use_tc_tiling_on_sc=True),
    scratch_types=[
        pltpu.VMEM((_NXB, _SUB, _D), jnp.float32),
        pltpu.VMEM((_NEB, _SUB, _D), jnp.float32),
        pltpu.SemaphoreType.DMA((_NXB,)),
        pltpu.SemaphoreType.DMA((_NXB,)),
        pltpu.SemaphoreType.DMA((_NEB,)),
    ],
)
def _sc_add(x_hbm, emb_hbm, out_hbm, xbuf, ebuf, xsem, osem, esem):
    wid = lax.axis_index("s") * _NC + lax.axis_index("c")
    s0 = wid * _SPW

    def rows(j):
        return pl.ds(s0 + j * _SUB, _SUB)

    def fire_x(t):
        j, b = divmod(t, _B)
        k = t % _NXB
        return pltpu.async_copy(x_hbm.at[b, rows(j)], xbuf.at[k], xsem.at[k])

    def fire_e(j):
        je = j % _NEB
        return pltpu.async_copy(emb_hbm.at[rows(j)], ebuf.at[je], esem.at[je])

    x_copies = {t: fire_x(t) for t in range(_AHEAD)}
    e_copies = {j: fire_e(j) for j in range(_NEB)}
    out_copies = {}

    for t in range(_STEPS):
        j, b = divmod(t, _B)
        k = t % _NXB
        if b == 0:
            e_copies.pop(j).wait()
        x_copies.pop(t).wait()
        t3 = t + _AHEAD
        if t3 < _STEPS:
            if t3 - _NXB in out_copies:
                out_copies.pop(t3 - _NXB).wait()
            x_copies[t3] = fire_x(t3)

        je = j % _NEB

        @plsc.parallel_loop(0, _VECS, unroll=8)
        def _add(i):
            r = i >> 6
            c = (i & 63) * 16
            plsc.addupdate(xbuf.at[k, r, pl.ds(c, 16)], ebuf[je, r, pl.ds(c, 16)])

        out_copies[t] = pltpu.async_copy(
            xbuf.at[k], out_hbm.at[b, rows(j)], osem.at[k])
        if b == _B - 1 and j + _NEB < _NSUB:
            e_copies[j + _NEB] = fire_e(j + _NEB)

    for t in sorted(out_copies):
        out_copies.pop(t).wait()


def kernel(x, emb):
    return _sc_add(x, emb)
